# baseline (speedup 1.0000x reference)
import functools
import math

import jax
import jax.numpy as jnp
import numpy as np
from jax.experimental import pallas as pl
from jax.experimental.pallas import tpu as pltpu

N_HEADS = 16
D_MODEL = 1024
D_HEAD = D_MODEL // N_HEADS
TOP_K = 64
CONTEXT_LEN = 2048
NEG_INF = -1e30


def _rope_tables_full(T, d_head):
    position = jnp.arange(T, dtype=jnp.float32)[:, None]
    div_term = 10000.0 ** (jnp.arange(0, d_head, 2, dtype=jnp.float32) / d_head)
    div_term = jnp.repeat(div_term, 2)
    cos = jnp.cos(position / div_term)
    sin = jnp.sin(position / div_term)
    return cos, sin


def _pair_swap_matrix(d_head):
    P = np.zeros((d_head, d_head), dtype=np.float32)
    for i in range(d_head // 2):
        P[2 * i + 1, 2 * i] = -1.0
        P[2 * i, 2 * i + 1] = 1.0
    return jnp.asarray(P)


def _encode(x):
    b = jax.lax.bitcast_convert_type(x, jnp.uint32)
    sign = jnp.uint32(0x80000000)
    return jnp.where(b >= sign, ~b, b | sign)


def _attn_kernel(cos_ref, sin_ref, perm_ref, q_ref, k_ref, v_ref, o_ref,
                 kr_ref, l_ref, u_ref, m_ref, cnt_ref, den_ref, z_ref,
                 acc_ref, *, bq, T, top_k, nchunks):
    qi = pl.program_id(1)
    scale = 1.0 / math.sqrt(D_HEAD)
    hi = jax.lax.Precision.HIGHEST

    P = perm_ref[...]

    @pl.when(qi == 0)
    def _():
        kh = k_ref[0]
        kr_ref[...] = kh * cos_ref[...] + jax.lax.dot(
            kh, P, preferred_element_type=jnp.float32, precision=hi
        ) * sin_ref[...]

    qh = q_ref[0]
    qpos = qi * bq
    cq = cos_ref[pl.ds(qpos, bq), :]
    sq = sin_ref[pl.ds(qpos, bq), :]
    qr = qh * cq + jax.lax.dot(
        qh, P, preferred_element_type=jnp.float32, precision=hi) * sq

    m_ref[...] = jnp.full((bq, 1), NEG_INF, jnp.float32)
    for c in range(nchunks):
        @pl.when(c <= qi)
        def _(c=c):
            kr = kr_ref[c * bq:(c + 1) * bq, :]
            lg = jax.lax.dot_general(
                qr, kr, (((1,), (1,)), ((), ())),
                preferred_element_type=jnp.float32) * scale
            row = qpos + jax.lax.broadcasted_iota(jnp.int32, (bq, bq), 0)
            col = c * bq + jax.lax.broadcasted_iota(jnp.int32, (bq, bq), 1)
            lg = jnp.where(col <= row, lg, NEG_INF)
            l_ref[:, c * bq:(c + 1) * bq] = lg
            u_ref[:, c * bq:(c + 1) * bq] = _encode(lg)
            m_ref[...] = jnp.maximum(m_ref[...], jnp.max(lg, axis=1,
                                                         keepdims=True))

    t = jnp.zeros((bq, 1), jnp.uint32)
    for i in range(31, -1, -1):
        cand = t | jnp.uint32(1 << i)
        cnt_ref[...] = jnp.zeros((bq, 1), jnp.float32)
        for c in range(nchunks):
            @pl.when(c <= qi)
            def _(c=c, cand=cand):
                u = u_ref[:, c * bq:(c + 1) * bq]
                cnt_ref[...] += jnp.sum((u >= cand).astype(jnp.float32),
                                        axis=1, keepdims=True)
        t = jnp.where(cnt_ref[...] >= float(top_k), cand, t)

    m = m_ref[...]
    den_ref[...] = jnp.zeros((bq, 1), jnp.float32)
    z_ref[...] = jnp.zeros((bq, 1), jnp.float32)
    acc_ref[...] = jnp.zeros((bq, D_HEAD), jnp.float32)
    for c in range(nchunks):
        @pl.when(c <= qi)
        def _(c=c, t=t):
            sl = slice(c * bq, (c + 1) * bq)
            e = jnp.exp(l_ref[:, sl] - m)
            w = jnp.where(u_ref[:, sl] >= t, e, 0.0)
            z_ref[...] += jnp.sum(e, axis=1, keepdims=True)
            den_ref[...] += jnp.sum(w, axis=1, keepdims=True)
            acc_ref[...] += jax.lax.dot(
                w, v_ref[0, sl, :], preferred_element_type=jnp.float32,
                precision=jax.lax.Precision.HIGHEST)

    denom = den_ref[...] + 1e-9 * z_ref[...]
    o_ref[0] = acc_ref[...] / denom


def kernel(q, k, v):
    b, T, d_model = q.shape
    H, d_head = N_HEADS, D_HEAD
    assert b == 1 and d_model == D_MODEL

    qh = q.reshape(T, H, d_head).transpose(1, 0, 2)
    kh = k.reshape(T, H, d_head).transpose(1, 0, 2)
    vh = v.reshape(T, H, d_head).transpose(1, 0, 2)

    cos, sin = _rope_tables_full(CONTEXT_LEN, d_head)
    cos = cos[:T]
    sin = sin[:T]
    P = _pair_swap_matrix(d_head)

    bq = min(256, T)
    nchunks = T // bq
    grid = (H, nchunks)

    out = pl.pallas_call(
        functools.partial(_attn_kernel, bq=bq, T=T, top_k=TOP_K,
                          nchunks=nchunks),
        grid=grid,
        in_specs=[
            pl.BlockSpec((T, d_head), lambda h, i: (0, 0)),
            pl.BlockSpec((T, d_head), lambda h, i: (0, 0)),
            pl.BlockSpec((d_head, d_head), lambda h, i: (0, 0)),
            pl.BlockSpec((1, bq, d_head), lambda h, i: (h, i, 0)),
            pl.BlockSpec((1, T, d_head), lambda h, i: (h, 0, 0)),
            pl.BlockSpec((1, T, d_head), lambda h, i: (h, 0, 0)),
        ],
        out_specs=pl.BlockSpec((1, bq, d_head), lambda h, i: (h, i, 0)),
        out_shape=jax.ShapeDtypeStruct((H, T, d_head), jnp.float32),
        scratch_shapes=[
            pltpu.VMEM((T, d_head), jnp.float32),
            pltpu.VMEM((bq, T), jnp.float32),
            pltpu.VMEM((bq, T), jnp.uint32),
            pltpu.VMEM((bq, 1), jnp.float32),
            pltpu.VMEM((bq, 1), jnp.float32),
            pltpu.VMEM((bq, 1), jnp.float32),
            pltpu.VMEM((bq, 1), jnp.float32),
            pltpu.VMEM((bq, D_HEAD), jnp.float32),
        ],
        compiler_params=pltpu.CompilerParams(
            dimension_semantics=("arbitrary", "arbitrary")),
    )(cos, sin, P, qh, kh, vh)

    return out.transpose(1, 0, 2).reshape(1, T, d_model)


# speedup vs baseline: 2.5849x; 2.5849x over previous
import functools
import math

import jax
import jax.numpy as jnp
import numpy as np
from jax.experimental import pallas as pl
from jax.experimental.pallas import tpu as pltpu

N_HEADS = 16
D_MODEL = 1024
D_HEAD = D_MODEL // N_HEADS
TOP_K = 64
CONTEXT_LEN = 2048
NEG_INF = -1e30


def _rope_tables_full(T, d_head):
    position = jnp.arange(T, dtype=jnp.float32)[:, None]
    div_term = 10000.0 ** (jnp.arange(0, d_head, 2, dtype=jnp.float32) / d_head)
    div_term = jnp.repeat(div_term, 2)
    cos = jnp.cos(position / div_term)
    sin = jnp.sin(position / div_term)
    return cos, sin


def _pair_swap_matrix(d_head):
    P = np.zeros((d_head, d_head), dtype=np.float32)
    for i in range(d_head // 2):
        P[2 * i + 1, 2 * i] = -1.0
        P[2 * i, 2 * i + 1] = 1.0
    return jnp.asarray(P)


def _encode(x):
    b = jax.lax.bitcast_convert_type(x, jnp.uint32)
    sign = jnp.uint32(0x80000000)
    return jnp.where(b >= sign, ~b, b | sign)


def _attn_kernel(cos_ref, sin_ref, perm_ref, q_ref, k_ref, v_ref, o_ref,
                 kr_ref, l_ref, u_ref, m_ref, cnt_ref, den_ref, z_ref,
                 acc_ref, *, bq, T, top_k, nchunks):
    qi = pl.program_id(1)
    scale = 1.0 / math.sqrt(D_HEAD)
    hi = jax.lax.Precision.HIGHEST

    P = perm_ref[...]

    @pl.when(qi == 0)
    def _():
        kh = k_ref[0]
        kr_ref[...] = kh * cos_ref[...] + jax.lax.dot(
            kh, P, preferred_element_type=jnp.float32, precision=hi
        ) * sin_ref[...]

    qh = q_ref[0]
    qpos = qi * bq
    cq = cos_ref[pl.ds(qpos, bq), :]
    sq = sin_ref[pl.ds(qpos, bq), :]
    qr = qh * cq + jax.lax.dot(
        qh, P, preferred_element_type=jnp.float32, precision=hi) * sq

    half = bq // 2
    for c in range(nchunks):
        @pl.when(c <= qi)
        def _(c=c):
            kr = kr_ref[c * bq:(c + 1) * bq, :]
            lg = jax.lax.dot_general(
                qr, kr, (((1,), (1,)), ((), ())),
                preferred_element_type=jnp.float32) * scale
            row = qpos + jax.lax.broadcasted_iota(jnp.int32, (bq, bq), 0)
            col = c * bq + jax.lax.broadcasted_iota(jnp.int32, (bq, bq), 1)
            lg = jnp.where(col <= row, lg, NEG_INF)
            l_ref[c] = lg
            u_ref[c] = _encode(lg)
            fold = jnp.maximum(lg[:, :half], lg[:, half:])
            if c == 0:
                m_ref[...] = fold
            else:
                m_ref[...] = jnp.maximum(m_ref[...], fold)
    m = jnp.max(m_ref[...], axis=1, keepdims=True)

    t = jnp.zeros((bq, 1), jnp.uint32)
    for i in range(31, -1, -1):
        cand = t | jnp.uint32(1 << i)
        for c in range(nchunks):
            @pl.when(c <= qi)
            def _(c=c, cand=cand):
                mk = (u_ref[c] >= cand).astype(jnp.float32)
                fold = mk[:, :half] + mk[:, half:]
                if c == 0:
                    cnt_ref[...] = fold
                else:
                    cnt_ref[...] += fold
        cnt = jnp.sum(cnt_ref[...], axis=1, keepdims=True)
        t = jnp.where(cnt >= float(top_k), cand, t)

    for c in range(nchunks):
        @pl.when(c <= qi)
        def _(c=c, t=t, m=m):
            e = jnp.exp(l_ref[c] - m)
            w = jnp.where(u_ref[c] >= t, e, 0.0)
            zf = e[:, :half] + e[:, half:]
            wf = w[:, :half] + w[:, half:]
            wv = jax.lax.dot(
                w, v_ref[0, c * bq:(c + 1) * bq, :],
                preferred_element_type=jnp.float32,
                precision=jax.lax.Precision.HIGHEST)
            if c == 0:
                z_ref[...] = zf
                den_ref[...] = wf
                acc_ref[...] = wv
            else:
                z_ref[...] += zf
                den_ref[...] += wf
                acc_ref[...] += wv

    den = jnp.sum(den_ref[...], axis=1, keepdims=True)
    z = jnp.sum(z_ref[...], axis=1, keepdims=True)
    denom = den + 1e-9 * z
    o_ref[0] = acc_ref[...] / denom


def kernel(q, k, v):
    b, T, d_model = q.shape
    H, d_head = N_HEADS, D_HEAD
    assert b == 1 and d_model == D_MODEL

    qh = q.reshape(T, H, d_head).transpose(1, 0, 2)
    kh = k.reshape(T, H, d_head).transpose(1, 0, 2)
    vh = v.reshape(T, H, d_head).transpose(1, 0, 2)

    cos, sin = _rope_tables_full(CONTEXT_LEN, d_head)
    cos = cos[:T]
    sin = sin[:T]
    P = _pair_swap_matrix(d_head)

    bq = min(256, T)
    nchunks = T // bq
    grid = (H, nchunks)

    out = pl.pallas_call(
        functools.partial(_attn_kernel, bq=bq, T=T, top_k=TOP_K,
                          nchunks=nchunks),
        grid=grid,
        in_specs=[
            pl.BlockSpec((T, d_head), lambda h, i: (0, 0)),
            pl.BlockSpec((T, d_head), lambda h, i: (0, 0)),
            pl.BlockSpec((d_head, d_head), lambda h, i: (0, 0)),
            pl.BlockSpec((1, bq, d_head), lambda h, i: (h, i, 0)),
            pl.BlockSpec((1, T, d_head), lambda h, i: (h, 0, 0)),
            pl.BlockSpec((1, T, d_head), lambda h, i: (h, 0, 0)),
        ],
        out_specs=pl.BlockSpec((1, bq, d_head), lambda h, i: (h, i, 0)),
        out_shape=jax.ShapeDtypeStruct((H, T, d_head), jnp.float32),
        scratch_shapes=[
            pltpu.VMEM((T, d_head), jnp.float32),
            pltpu.VMEM((nchunks, bq, bq), jnp.float32),
            pltpu.VMEM((nchunks, bq, bq), jnp.uint32),
            pltpu.VMEM((bq, bq // 2), jnp.float32),
            pltpu.VMEM((bq, bq // 2), jnp.float32),
            pltpu.VMEM((bq, bq // 2), jnp.float32),
            pltpu.VMEM((bq, bq // 2), jnp.float32),
            pltpu.VMEM((bq, D_HEAD), jnp.float32),
        ],
        compiler_params=pltpu.CompilerParams(
            dimension_semantics=("arbitrary", "arbitrary")),
    )(cos, sin, P, qh, kh, vh)

    return out.transpose(1, 0, 2).reshape(1, T, d_model)
